# baseline (speedup 1.0000x reference)
import functools

import jax
import jax.numpy as jnp
from jax import lax
from jax.experimental import pallas as pl
from jax.experimental.pallas import tpu as pltpu
from jax.experimental.pallas import tpu_sc as plsc

_NC = 2
_NS = 16
_NW = _NC * _NS
_V = 21
_NP = _V * _V
_D = 80
_DP = 128
_CP = 128
_NSPLIT = 4
_RB = 256


def _sc_gather_pairs(pidx2d, tpairs, BPS):
    p_per_w = BPS // _NW
    n_chunks = p_per_w // _CP
    assert p_per_w % _CP == 0 and n_chunks % 2 == 0

    mesh = plsc.VectorSubcoreMesh(core_axis_name="c", subcore_axis_name="s")

    @functools.partial(
        pl.kernel,
        mesh=mesh,
        out_type=jax.ShapeDtypeStruct((BPS, 2 * _DP), jnp.float32),
        scratch_types=[
            pltpu.VMEM((2, 1, _CP), jnp.int32),
            pltpu.VMEM((2, _CP, 2 * _DP), jnp.float32),
            pltpu.SemaphoreType.DMA,
            pltpu.SemaphoreType.DMA,
            pltpu.SemaphoreType.DMA,
            pltpu.SemaphoreType.DMA,
        ],
    )
    def k(pidx_hbm, tpairs_hbm, out_hbm, idx_v, rows_v, sg0, sg1, ss0, ss1):
        wid = lax.axis_index("s") * _NC + lax.axis_index("c")
        wrow0 = wid * n_chunks
        wbase = wid * p_per_w
        off = wid * _NP
        nloop = n_chunks // 2

        def stage_idx(g, slot):
            pltpu.sync_copy(pidx_hbm.at[pl.ds(wrow0 + g, 1)], idx_v.at[slot])
            for q in range(_CP // 16):
                sl = idx_v.at[slot, 0][pl.ds(q * 16, 16)]
                idx_v.at[slot, 0][pl.ds(q * 16, 16)] = sl + off

        def fire_gather(slot, sem):
            return pltpu.async_copy(
                tpairs_hbm.at[idx_v.at[slot, 0]], rows_v.at[slot], sem
            )

        def fire_scatter(g, slot, sem):
            return pltpu.async_copy(
                rows_v.at[slot], out_hbm.at[pl.ds(wbase + g * _CP, _CP)], sem
            )

        def drain_odd_scatter():
            pltpu.make_async_copy(
                rows_v.at[1], out_hbm.at[pl.ds(wbase, _CP)], ss1
            ).wait()

        def body(p, carry):
            g0 = 2 * p
            stage_idx(g0, 0)
            gcp = fire_gather(0, sg0)

            @pl.when(p >= 1)
            def _():
                drain_odd_scatter()

            gcp.wait()
            sc0 = fire_scatter(g0, 0, ss0)

            stage_idx(g0 + 1, 1)
            fire_gather(1, sg1).wait()
            sc0.wait()
            fire_scatter(g0 + 1, 1, ss1)
            return carry

        lax.fori_loop(0, nloop, body, 0)
        drain_odd_scatter()

    return k(pidx2d, tpairs)


def _depad_body(in_ref, out_ref):
    x = in_ref[...]
    out_ref[...] = x.reshape(_RB, 2, _DP)[:, :, : _D].reshape(2 * _RB, _D)


def _tc_depad(h, B, blk_off):
    BPS = h.shape[0]
    grid = BPS // _RB
    return pl.pallas_call(
        _depad_body,
        grid=(grid,),
        in_specs=[pl.BlockSpec((_RB, 2 * _DP), lambda i: (i, 0))],
        out_specs=pl.BlockSpec((2 * _RB, _D), lambda i, o=blk_off: (i + o, 0)),
        out_shape=jax.ShapeDtypeStruct((B, _D), jnp.float32),
    )(h)


def _tc_depad_into(acc, h, blk_off):
    B = acc.shape[0]
    BPS = h.shape[0]
    grid = BPS // _RB

    def body(acc_ref, in_ref, out_ref):
        _depad_body(in_ref, out_ref)

    return pl.pallas_call(
        body,
        grid=(grid,),
        in_specs=[
            pl.BlockSpec(memory_space=pl.ANY),
            pl.BlockSpec((_RB, 2 * _DP), lambda i: (i, 0)),
        ],
        out_specs=pl.BlockSpec((2 * _RB, _D), lambda i, o=blk_off: (i + o, 0)),
        out_shape=jax.ShapeDtypeStruct((B, _D), jnp.float32),
        input_output_aliases={0: 0},
    )(acc, h)


@functools.partial(jax.jit, static_argnames=("B",))
def _embed(pidx2d, tpairs, B):
    BP = B // 2
    BPS = BP // _NSPLIT
    rows_per_split = pidx2d.shape[0] // _NSPLIT
    halves = [
        _sc_gather_pairs(
            lax.slice_in_dim(pidx2d, i * rows_per_split, (i + 1) * rows_per_split),
            tpairs,
            BPS,
        )
        for i in range(_NSPLIT)
    ]
    blk_per_split = 2 * BPS // (2 * _RB)
    acc = _tc_depad(halves[0], B, 0)
    for i in range(1, _NSPLIT):
        acc = _tc_depad_into(acc, halves[i], i * blk_per_split)
    return acc


def kernel(residue_types, table):
    S, T = residue_types.shape
    B = S * T
    BP = B // 2
    pidx = residue_types.reshape(BP, 2)
    pidx2d = (pidx[:, 0] * _V + pidx[:, 1]).reshape(BP // _CP, _CP)
    tpad = jnp.pad(table, ((0, 0), (0, _DP - _D)))
    tp = jnp.concatenate(
        [
            jnp.broadcast_to(tpad[:, None, :], (_V, _V, _DP)).reshape(_NP, _DP),
            jnp.broadcast_to(tpad[None, :, :], (_V, _V, _DP)).reshape(_NP, _DP),
        ],
        axis=1,
    )
    tpairs = jnp.tile(tp, (_NW, 1))
    out = _embed(pidx2d, tpairs, B)
    return out.reshape(S, T, _D)


# speedup vs baseline: 1.4729x; 1.4729x over previous
import functools

import jax
import jax.numpy as jnp
from jax import lax
from jax.experimental import pallas as pl
from jax.experimental.pallas import tpu as pltpu
from jax.experimental.pallas import tpu_sc as plsc

_NC = 2
_NS = 16
_NW = _NC * _NS
_V = 21
_NP = _V * _V
_D = 80
_DP = 128
_CP = 128


@functools.partial(jax.jit, static_argnames=("BP",))
def _sc_embed(pidx2d, tpairs, BP):
    p_per_w = BP // _NW
    n_chunks = p_per_w // _CP
    assert p_per_w % _CP == 0 and n_chunks % 2 == 0

    mesh = plsc.VectorSubcoreMesh(core_axis_name="c", subcore_axis_name="s")

    @functools.partial(
        pl.kernel,
        mesh=mesh,
        out_type=jax.ShapeDtypeStruct((BP, 2 * _DP), jnp.float32),
        scratch_types=[
            pltpu.VMEM((2, 1, _CP), jnp.int32),
            pltpu.VMEM((2, _CP, 2 * _DP), jnp.float32),
            pltpu.SemaphoreType.DMA,
            pltpu.SemaphoreType.DMA,
            pltpu.SemaphoreType.DMA,
            pltpu.SemaphoreType.DMA,
        ],
    )
    def k(pidx_hbm, tpairs_hbm, out_hbm, idx_v, rows_v, sg0, sg1, ss0, ss1):
        wid = lax.axis_index("s") * _NC + lax.axis_index("c")
        wrow0 = wid * n_chunks
        wbase = wid * p_per_w
        off = wid * _NP
        nloop = n_chunks // 2

        def stage_idx(g, slot):
            pltpu.sync_copy(pidx_hbm.at[pl.ds(wrow0 + g, 1)], idx_v.at[slot])
            for q in range(_CP // 16):
                sl = idx_v.at[slot, 0][pl.ds(q * 16, 16)]
                idx_v.at[slot, 0][pl.ds(q * 16, 16)] = sl + off

        def fire_gather(slot, sem):
            return pltpu.async_copy(
                tpairs_hbm.at[idx_v.at[slot, 0]], rows_v.at[slot], sem
            )

        def fire_scatter(g, slot, sem):
            return pltpu.async_copy(
                rows_v.at[slot], out_hbm.at[pl.ds(wbase + g * _CP, _CP)], sem
            )

        def drain_odd_scatter():
            pltpu.make_async_copy(
                rows_v.at[1], out_hbm.at[pl.ds(wbase, _CP)], ss1
            ).wait()

        def body(p, carry):
            g0 = 2 * p
            stage_idx(g0, 0)
            gcp = fire_gather(0, sg0)

            @pl.when(p >= 1)
            def _():
                drain_odd_scatter()

            gcp.wait()
            sc0 = fire_scatter(g0, 0, ss0)

            stage_idx(g0 + 1, 1)
            fire_gather(1, sg1).wait()
            sc0.wait()
            fire_scatter(g0 + 1, 1, ss1)
            return carry

        lax.fori_loop(0, nloop, body, 0)
        drain_odd_scatter()

    return k(pidx2d, tpairs)


def kernel(residue_types, table):
    S, T = residue_types.shape
    B = S * T
    BP = B // 2
    pidx = residue_types.reshape(BP, 2)
    pidx2d = (pidx[:, 0] * _V + pidx[:, 1]).reshape(BP // _CP, _CP)
    tpad = jnp.pad(table, ((0, 0), (0, _DP - _D)))
    tpairs = jnp.tile(
        jnp.concatenate(
            [
                jnp.broadcast_to(tpad[:, None, :], (_V, _V, _DP)).reshape(_NP, _DP),
                jnp.broadcast_to(tpad[None, :, :], (_V, _V, _DP)).reshape(_NP, _DP),
            ],
            axis=1,
        ),
        (_NW, 1),
    )
    out = _sc_embed(pidx2d, tpairs, BP)
    return out.reshape(B, _DP)[:, :_D].reshape(S, T, _D)


# speedup vs baseline: 2.2464x; 1.5252x over previous
import functools

import jax
import jax.numpy as jnp
from jax import lax
from jax.experimental import pallas as pl
from jax.experimental.pallas import tpu as pltpu
from jax.experimental.pallas import tpu_sc as plsc

_NC = 2
_NS = 16
_NW = _NC * _NS
_D = 80
_DP = 128
_C = 256
_RJ = _C // 128


@functools.partial(jax.jit, static_argnames=("B",))
def _sc_embed(idx2d, table, B):
    b_per_w = B // _NW
    n_chunks = b_per_w // _C
    assert b_per_w % _C == 0 and n_chunks % 2 == 0

    mesh = plsc.VectorSubcoreMesh(core_axis_name="c", subcore_axis_name="s")

    @functools.partial(
        pl.kernel,
        mesh=mesh,
        out_type=jax.ShapeDtypeStruct((B, _DP), jnp.float32),
        scratch_types=[
            pltpu.VMEM((2, _RJ, 128), jnp.int32),
            pltpu.VMEM((2, _C, _DP), jnp.float32),
            pltpu.SemaphoreType.DMA,
            pltpu.SemaphoreType.DMA,
            pltpu.SemaphoreType.DMA,
            pltpu.SemaphoreType.DMA,
        ],
    )
    def k(idx_hbm, table_hbm, out_hbm, idx_v, rows_v, sg0, sg1, ss0, ss1):
        wid = lax.axis_index("s") * _NC + lax.axis_index("c")
        wrow0 = wid * (b_per_w // 128)
        wbase = wid * b_per_w
        off = wid * 21
        npairs = n_chunks // 2

        def stage_idx(g, slot):
            pltpu.sync_copy(idx_hbm.at[pl.ds(wrow0 + g * _RJ, _RJ)], idx_v.at[slot])
            for j in range(_RJ):
                for q in range(128 // 16):
                    sl = idx_v.at[slot, j][pl.ds(q * 16, 16)]
                    idx_v.at[slot, j][pl.ds(q * 16, 16)] = sl + off

        def fire_gathers(slot, sem):
            return [
                pltpu.async_copy(
                    table_hbm.at[idx_v.at[slot, j]],
                    rows_v.at[slot, pl.ds(j * 128, 128)],
                    sem,
                )
                for j in range(_RJ)
            ]

        def fire_scatter(g, slot, sem):
            return pltpu.async_copy(
                rows_v.at[slot],
                out_hbm.at[pl.ds(wbase + g * _C, _C)],
                sem,
            )

        def drain_odd_scatter():
            pltpu.make_async_copy(
                rows_v.at[1], out_hbm.at[pl.ds(wbase, _C)], ss1
            ).wait()

        def body(p, carry):
            g0 = 2 * p
            stage_idx(g0, 0)
            g_cps = fire_gathers(0, sg0)

            @pl.when(p >= 1)
            def _():
                drain_odd_scatter()

            for cp in g_cps:
                cp.wait()
            sc0 = fire_scatter(g0, 0, ss0)

            stage_idx(g0 + 1, 1)
            for cp in fire_gathers(1, sg1):
                cp.wait()
            sc0.wait()
            fire_scatter(g0 + 1, 1, ss1)
            return carry

        lax.fori_loop(0, npairs, body, 0)
        drain_odd_scatter()

    return k(idx2d, table)


def kernel(residue_types, table):
    S, T = residue_types.shape
    B = S * T
    idx2d = residue_types.reshape(B // 128, 128)
    table_rep = jnp.tile(jnp.pad(table, ((0, 0), (0, _DP - _D))), (_NW, 1))
    out = _sc_embed(idx2d, table_rep, B)
    return out[:, :_D].reshape(S, T, _D)


# speedup vs baseline: 2.3264x; 1.0356x over previous
import functools

import jax
import jax.numpy as jnp
from jax import lax
from jax.experimental import pallas as pl
from jax.experimental.pallas import tpu as pltpu
from jax.experimental.pallas import tpu_sc as plsc

_NC = 2
_NS = 16
_NW = _NC * _NS
_D = 80
_DP = 128
_C = 128
_NSLOT = 4


@functools.partial(jax.jit, static_argnames=("B",))
def _sc_embed(idx2d, table, B):
    b_per_w = B // _NW
    n_chunks = b_per_w // _C
    assert b_per_w % _C == 0 and n_chunks % _NSLOT == 0

    mesh = plsc.VectorSubcoreMesh(core_axis_name="c", subcore_axis_name="s")

    @functools.partial(
        pl.kernel,
        mesh=mesh,
        out_type=jax.ShapeDtypeStruct((B, _DP), jnp.float32),
        scratch_types=[
            pltpu.VMEM((_NSLOT, 1, _C), jnp.int32),
            pltpu.VMEM((_NSLOT, _C, _DP), jnp.float32),
        ]
        + [pltpu.SemaphoreType.DMA] * (2 * _NSLOT),
    )
    def k(idx_hbm, table_hbm, out_hbm, idx_v, rows_v, *sems):
        sg = sems[:_NSLOT]
        ss = sems[_NSLOT:]
        wid = lax.axis_index("s") * _NC + lax.axis_index("c")
        wrow0 = wid * n_chunks
        wbase = wid * b_per_w
        off = wid * 21
        ngroups = n_chunks // _NSLOT

        def stage_idx(g, slot):
            pltpu.sync_copy(idx_hbm.at[pl.ds(wrow0 + g, 1)], idx_v.at[slot])
            for q in range(_C // 16):
                sl = idx_v.at[slot, 0][pl.ds(q * 16, 16)]
                idx_v.at[slot, 0][pl.ds(q * 16, 16)] = sl + off

        def drain_scatter(slot):
            pltpu.make_async_copy(
                rows_v.at[slot], out_hbm.at[pl.ds(wbase, _C)], ss[slot]
            ).wait()

        def body(p, carry):
            g0 = _NSLOT * p
            gcps = []
            for s in range(_NSLOT):

                @pl.when(p >= 1)
                def _(s=s):
                    drain_scatter(s)

                stage_idx(g0 + s, s)
                gcps.append(
                    pltpu.async_copy(
                        table_hbm.at[idx_v.at[s, 0]], rows_v.at[s], sg[s]
                    )
                )
            for s in range(_NSLOT):
                gcps[s].wait()
                pltpu.async_copy(
                    rows_v.at[s],
                    out_hbm.at[pl.ds(wbase + (g0 + s) * _C, _C)],
                    ss[s],
                )
            return carry

        lax.fori_loop(0, ngroups, body, 0)
        for s in range(_NSLOT):
            drain_scatter(s)

    return k(idx2d, table)


def kernel(residue_types, table):
    S, T = residue_types.shape
    B = S * T
    idx2d = residue_types.reshape(B // 128, 128)
    table_rep = jnp.tile(jnp.pad(table, ((0, 0), (0, _DP - _D))), (_NW, 1))
    out = _sc_embed(idx2d, table_rep, B)
    return out[:, :_D].reshape(S, T, _D)


# speedup vs baseline: 2.3453x; 1.0082x over previous
import functools

import jax
import jax.numpy as jnp
from jax import lax
from jax.experimental import pallas as pl
from jax.experimental.pallas import tpu as pltpu
from jax.experimental.pallas import tpu_sc as plsc

_NC = 2
_NS = 16
_NW = _NC * _NS
_D = 80
_DP = 128
_C = 128
_NSLOT = 5


@functools.partial(jax.jit, static_argnames=("B",))
def _sc_embed(idx2d, table, B):
    b_per_w = B // _NW
    n_chunks = b_per_w // _C
    assert b_per_w % _C == 0 and n_chunks % _NSLOT == 0

    mesh = plsc.VectorSubcoreMesh(core_axis_name="c", subcore_axis_name="s")

    @functools.partial(
        pl.kernel,
        mesh=mesh,
        out_type=jax.ShapeDtypeStruct((B, _DP), jnp.float32),
        scratch_types=[
            pltpu.VMEM((_NSLOT, 1, _C), jnp.int32),
            pltpu.VMEM((_NSLOT, _C, _DP), jnp.float32),
        ]
        + [pltpu.SemaphoreType.DMA] * (2 * _NSLOT),
    )
    def k(idx_hbm, table_hbm, out_hbm, idx_v, rows_v, *sems):
        sg = sems[:_NSLOT]
        ss = sems[_NSLOT:]
        wid = lax.axis_index("s") * _NC + lax.axis_index("c")
        wrow0 = wid * n_chunks
        wbase = wid * b_per_w
        off = wid * 21
        ngroups = n_chunks // _NSLOT

        def stage_idx(g, slot):
            pltpu.sync_copy(idx_hbm.at[pl.ds(wrow0 + g, 1)], idx_v.at[slot])
            for q in range(_C // 16):
                sl = idx_v.at[slot, 0][pl.ds(q * 16, 16)]
                idx_v.at[slot, 0][pl.ds(q * 16, 16)] = sl + off

        def drain_scatter(slot):
            pltpu.make_async_copy(
                rows_v.at[slot], out_hbm.at[pl.ds(wbase, _C)], ss[slot]
            ).wait()

        def body(p, carry):
            g0 = _NSLOT * p
            gcps = []
            for s in range(_NSLOT):

                @pl.when(p >= 1)
                def _(s=s):
                    drain_scatter(s)

                stage_idx(g0 + s, s)
                gcps.append(
                    pltpu.async_copy(
                        table_hbm.at[idx_v.at[s, 0]], rows_v.at[s], sg[s]
                    )
                )
            for s in range(_NSLOT):
                gcps[s].wait()
                pltpu.async_copy(
                    rows_v.at[s],
                    out_hbm.at[pl.ds(wbase + (g0 + s) * _C, _C)],
                    ss[s],
                )
            return carry

        lax.fori_loop(0, ngroups, body, 0)
        for s in range(_NSLOT):
            drain_scatter(s)

    return k(idx2d, table)


def kernel(residue_types, table):
    S, T = residue_types.shape
    B = S * T
    idx2d = residue_types.reshape(B // 128, 128)
    table_rep = jnp.tile(jnp.pad(table, ((0, 0), (0, _DP - _D))), (_NW, 1))
    out = _sc_embed(idx2d, table_rep, B)
    return out[:, :_D].reshape(S, T, _D)
